# baseline (speedup 1.0000x reference)
import functools

import jax
import jax.numpy as jnp
from jax.experimental import pallas as pl
from jax.experimental.pallas import tpu as pltpu

N_MAX = 256
N_HEADS = 4
D_HEAD = 128
ALPHA = 0.08838834764831843
ROW_F = N_HEADS * 3 * D_HEAD
OUT_F = N_HEADS * D_HEAD
HALF = N_MAX // 2
PANEL = N_MAX + HALF


def _silu_mask(s, m):
    a = s * jax.nn.sigmoid(s)
    return jnp.where(m, a, 0.0).astype(jnp.bfloat16)


def _block_kernel(rec_ref, rep_ref, xp_ref, xc_ref, o_ref, *, lp_rows):
    b = pl.program_id(0)
    base = b * N_MAX

    xc = xc_ref[...]
    xp = xp_ref[...]

    upper = jnp.concatenate([rep_ref[0], rec_ref[0]], axis=1) - base

    gi = jax.lax.broadcasted_iota(jnp.uint32, (HALF, PANEL), 0)
    ci = jax.lax.broadcasted_iota(jnp.uint32, (HALF, PANEL), 1)
    d = gi + N_MAX - ci
    cw = jax.lax.broadcasted_iota(jnp.int32, (1, 2 * N_MAX), 1) - N_MAX
    w = jnp.maximum(upper - cw, 0).astype(jnp.uint32)
    mask_t = d < w[:, :PANEL]
    mask_b = d < w[:, HALF:]

    vrow = jax.lax.broadcasted_iota(jnp.int32, (2 * N_MAX, D_HEAD), 0)
    vg = vrow + (base - N_MAX)
    vok = (vg >= 0) & (vg < lp_rows)

    alpha = jnp.bfloat16(ALPHA)
    vscale = jnp.bfloat16(1.0 / N_MAX)

    for h in range(N_HEADS):
        o = h * 3 * D_HEAD
        q = xc[:, o:o + D_HEAD] * alpha
        k = jnp.concatenate(
            [xp[:, o + D_HEAD:o + 2 * D_HEAD], xc[:, o + D_HEAD:o + 2 * D_HEAD]],
            axis=0)
        v = jnp.concatenate(
            [xp[:, o + 2 * D_HEAD:o + 3 * D_HEAD], xc[:, o + 2 * D_HEAD:o + 3 * D_HEAD]],
            axis=0)
        v = jnp.where(vok, v * vscale, jnp.bfloat16(0))

        s_t = jax.lax.dot_general(q[:HALF], k[:PANEL],
                                  (((1,), (1,)), ((), ())),
                                  preferred_element_type=jnp.float32)
        o_t = jax.lax.dot_general(_silu_mask(s_t, mask_t), v[:PANEL],
                                  (((1,), (0,)), ((), ())),
                                  preferred_element_type=jnp.float32)

        s_b = jax.lax.dot_general(q[HALF:], k[HALF:],
                                  (((1,), (1,)), ((), ())),
                                  preferred_element_type=jnp.float32)
        o_b = jax.lax.dot_general(_silu_mask(s_b, mask_b), v[HALF:],
                                  (((1,), (0,)), ((), ())),
                                  preferred_element_type=jnp.float32)

        hs = h * D_HEAD
        o_ref[:HALF, hs:hs + D_HEAD] = o_t.astype(jnp.bfloat16)
        o_ref[HALF:, hs:hs + D_HEAD] = o_b.astype(jnp.bfloat16)


@jax.jit
def kernel(qkv, seq_offsets, timestamps, tw, pw):
    L = qkv.shape[0]
    nb = (L + N_MAX - 1) // N_MAX
    x = qkv.reshape(L, ROW_F)

    offs = seq_offsets.astype(jnp.int32)
    lengths = offs[1:] - offs[:-1]
    row_end = jnp.repeat(offs[1:], lengths, total_repeat_length=L)
    row_end = jnp.pad(row_end, (0, nb * N_MAX - L))
    re3 = row_end.reshape(nb, 1, N_MAX)
    rp3 = jnp.pad(re3[:-1], ((1, 0), (0, 0), (0, 0)))

    out = pl.pallas_call(
        functools.partial(_block_kernel, lp_rows=L),
        grid=(nb,),
        in_specs=[
            pl.BlockSpec((1, 1, N_MAX), lambda b: (b, 0, 0)),
            pl.BlockSpec((1, 1, N_MAX), lambda b: (b, 0, 0)),
            pl.BlockSpec((N_MAX, ROW_F), lambda b: (jnp.maximum(b - 1, 0), 0)),
            pl.BlockSpec((N_MAX, ROW_F), lambda b: (b, 0)),
        ],
        out_specs=pl.BlockSpec((N_MAX, OUT_F), lambda b: (b, 0)),
        out_shape=jax.ShapeDtypeStruct((L, OUT_F), jnp.bfloat16),
        compiler_params=pltpu.CompilerParams(
            dimension_semantics=("parallel",),
        ),
    )(re3, rp3, x, x)
    return out.reshape(L, N_HEADS, D_HEAD)


# speedup vs baseline: 1.0635x; 1.0635x over previous
import functools

import jax
import jax.numpy as jnp
from jax.experimental import pallas as pl
from jax.experimental.pallas import tpu as pltpu

N_MAX = 256
N_HEADS = 4
D_HEAD = 128
ALPHA = 0.08838834764831843
ROW_F = N_HEADS * 3 * D_HEAD
OUT_F = N_HEADS * D_HEAD
HALF = N_MAX // 2
PANEL = N_MAX + HALF


def _silu_mask(s, m):
    a = s * jax.nn.sigmoid(s)
    return jnp.where(m, a, 0.0).astype(jnp.bfloat16)


def _block_kernel(rec_ref, rep_ref, xp_ref, xc_ref, o_ref, *, lp_rows):
    b = pl.program_id(0)
    base = b * N_MAX

    xc = xc_ref[...]
    xp = xp_ref[...]

    upper = jnp.concatenate([rep_ref[0], rec_ref[0]], axis=1) - base

    gi = jax.lax.broadcasted_iota(jnp.uint32, (N_MAX, 2 * N_MAX), 0)
    ci = jax.lax.broadcasted_iota(jnp.uint32, (N_MAX, 2 * N_MAX), 1)
    d = gi + N_MAX - ci
    cw = jax.lax.broadcasted_iota(jnp.int32, (1, 2 * N_MAX), 1) - N_MAX
    w = jnp.maximum(upper - cw, 0).astype(jnp.uint32)
    mask = d < w

    vrow = jax.lax.broadcasted_iota(jnp.int32, (2 * N_MAX, D_HEAD), 0)
    vg = vrow + (base - N_MAX)
    vok = (vg >= 0) & (vg < lp_rows)

    alpha = jnp.bfloat16(ALPHA)
    vscale = jnp.bfloat16(1.0 / N_MAX)

    for h in range(N_HEADS):
        o = h * 3 * D_HEAD
        q = xc[:, o:o + D_HEAD] * alpha
        k = jnp.concatenate(
            [xp[:, o + D_HEAD:o + 2 * D_HEAD], xc[:, o + D_HEAD:o + 2 * D_HEAD]],
            axis=0)
        v = jnp.concatenate(
            [xp[:, o + 2 * D_HEAD:o + 3 * D_HEAD], xc[:, o + 2 * D_HEAD:o + 3 * D_HEAD]],
            axis=0)
        v = jnp.where(vok, v * vscale, jnp.bfloat16(0))

        s = jax.lax.dot_general(q, k,
                                (((1,), (1,)), ((), ())),
                                preferred_element_type=jnp.float32)
        ov = jax.lax.dot_general(_silu_mask(s, mask), v,
                                 (((1,), (0,)), ((), ())),
                                 preferred_element_type=jnp.float32)

        hs = h * D_HEAD
        o_ref[:, hs:hs + D_HEAD] = ov.astype(jnp.bfloat16)


@jax.jit
def kernel(qkv, seq_offsets, timestamps, tw, pw):
    L = qkv.shape[0]
    nb = (L + N_MAX - 1) // N_MAX
    x = qkv.reshape(L, ROW_F)

    offs = seq_offsets.astype(jnp.int32)
    lengths = offs[1:] - offs[:-1]
    row_end = jnp.repeat(offs[1:], lengths, total_repeat_length=L)
    row_end = jnp.pad(row_end, (0, nb * N_MAX - L))
    re3 = row_end.reshape(nb, 1, N_MAX)
    rp3 = jnp.pad(re3[:-1], ((1, 0), (0, 0), (0, 0)))

    out = pl.pallas_call(
        functools.partial(_block_kernel, lp_rows=L),
        grid=(nb,),
        in_specs=[
            pl.BlockSpec((1, 1, N_MAX), lambda b: (b, 0, 0)),
            pl.BlockSpec((1, 1, N_MAX), lambda b: (b, 0, 0)),
            pl.BlockSpec((N_MAX, ROW_F), lambda b: (jnp.maximum(b - 1, 0), 0)),
            pl.BlockSpec((N_MAX, ROW_F), lambda b: (b, 0)),
        ],
        out_specs=pl.BlockSpec((N_MAX, OUT_F), lambda b: (b, 0)),
        out_shape=jax.ShapeDtypeStruct((L, OUT_F), jnp.bfloat16),
        compiler_params=pltpu.CompilerParams(
            dimension_semantics=("parallel",),
        ),
    )(re3, rp3, x, x)
    return out.reshape(L, N_HEADS, D_HEAD)


# speedup vs baseline: 1.0678x; 1.0040x over previous
import functools

import jax
import jax.numpy as jnp
from jax.experimental import pallas as pl
from jax.experimental.pallas import tpu as pltpu

N_MAX = 256
N_HEADS = 4
D_HEAD = 128
ALPHA = 0.08838834764831843
ROW_F = N_HEADS * 3 * D_HEAD
OUT_F = N_HEADS * D_HEAD
HALF = N_MAX // 2
PANEL = N_MAX + HALF


def _silu_mask(s, m):
    sb = s.astype(jnp.bfloat16)
    a = sb * jax.nn.sigmoid(sb)
    return jnp.where(m, a, jnp.bfloat16(0))


def _block_kernel(rec_ref, rep_ref, xp_ref, xc_ref, o_ref, *, lp_rows):
    b = pl.program_id(0)
    base = b * N_MAX

    xc = xc_ref[...]
    xp = xp_ref[...]

    upper = jnp.concatenate([rep_ref[0], rec_ref[0]], axis=1) - base

    gi = jax.lax.broadcasted_iota(jnp.uint32, (N_MAX, 2 * N_MAX), 0)
    ci = jax.lax.broadcasted_iota(jnp.uint32, (N_MAX, 2 * N_MAX), 1)
    d = gi + N_MAX - ci
    cw = jax.lax.broadcasted_iota(jnp.int32, (1, 2 * N_MAX), 1) - N_MAX
    w = jnp.maximum(upper - cw, 0).astype(jnp.uint32)
    mask = d < w

    vrow = jax.lax.broadcasted_iota(jnp.int32, (2 * N_MAX, D_HEAD), 0)
    vg = vrow + (base - N_MAX)
    vok = (vg >= 0) & (vg < lp_rows)

    alpha = jnp.bfloat16(ALPHA)
    vscale = jnp.bfloat16(1.0 / N_MAX)

    for h in range(N_HEADS):
        o = h * 3 * D_HEAD
        q = xc[:, o:o + D_HEAD] * alpha
        k = jnp.concatenate(
            [xp[:, o + D_HEAD:o + 2 * D_HEAD], xc[:, o + D_HEAD:o + 2 * D_HEAD]],
            axis=0)
        v = jnp.concatenate(
            [xp[:, o + 2 * D_HEAD:o + 3 * D_HEAD], xc[:, o + 2 * D_HEAD:o + 3 * D_HEAD]],
            axis=0)
        v = jnp.where(vok, v * vscale, jnp.bfloat16(0))

        s = jax.lax.dot_general(q, k,
                                (((1,), (1,)), ((), ())),
                                preferred_element_type=jnp.float32)
        ov = jax.lax.dot_general(_silu_mask(s, mask), v,
                                 (((1,), (0,)), ((), ())),
                                 preferred_element_type=jnp.float32)

        hs = h * D_HEAD
        o_ref[:, hs:hs + D_HEAD] = ov.astype(jnp.bfloat16)


@jax.jit
def kernel(qkv, seq_offsets, timestamps, tw, pw):
    L = qkv.shape[0]
    nb = (L + N_MAX - 1) // N_MAX
    x = qkv.reshape(L, ROW_F)

    offs = seq_offsets.astype(jnp.int32)
    lengths = offs[1:] - offs[:-1]
    row_end = jnp.repeat(offs[1:], lengths, total_repeat_length=L)
    row_end = jnp.pad(row_end, (0, nb * N_MAX - L))
    re3 = row_end.reshape(nb, 1, N_MAX)
    rp3 = jnp.pad(re3[:-1], ((1, 0), (0, 0), (0, 0)))

    out = pl.pallas_call(
        functools.partial(_block_kernel, lp_rows=L),
        grid=(nb,),
        in_specs=[
            pl.BlockSpec((1, 1, N_MAX), lambda b: (b, 0, 0)),
            pl.BlockSpec((1, 1, N_MAX), lambda b: (b, 0, 0)),
            pl.BlockSpec((N_MAX, ROW_F), lambda b: (jnp.maximum(b - 1, 0), 0)),
            pl.BlockSpec((N_MAX, ROW_F), lambda b: (b, 0)),
        ],
        out_specs=pl.BlockSpec((N_MAX, OUT_F), lambda b: (b, 0)),
        out_shape=jax.ShapeDtypeStruct((L, OUT_F), jnp.bfloat16),
        compiler_params=pltpu.CompilerParams(
            dimension_semantics=("parallel",),
        ),
    )(re3, rp3, x, x)
    return out.reshape(L, N_HEADS, D_HEAD)
